# baseline (speedup 1.0000x reference)
import functools

import jax
import jax.numpy as jnp
from jax import lax
from jax.experimental import pallas as pl
from jax.experimental.pallas import tpu as pltpu
from jax.experimental.pallas import tpu_sc as plsc

NW = 32
CH = 128
TB = 256
D = 64


def _sc_gather(table, idx):
    n = idx.shape[0]
    per_w = n // NW
    nch = per_w // CH
    idx3 = idx.reshape(NW, nch, CH)
    mesh = plsc.VectorSubcoreMesh(core_axis_name="c", subcore_axis_name="s")

    @functools.partial(
        pl.kernel,
        mesh=mesh,
        out_type=jax.ShapeDtypeStruct((n, D), jnp.float32),
        compiler_params=pltpu.CompilerParams(use_tc_tiling_on_sc=False),
        scratch_types=[
            pltpu.VMEM((nch, CH), jnp.int32),
            pltpu.VMEM((per_w, D), jnp.float32),
            pltpu.SemaphoreType.DMA,
        ],
    )
    def k(table_hbm, idx_hbm, out_hbm, idx_v, rows_v, sem):
        wid = lax.axis_index("s") * 2 + lax.axis_index("c")
        base = wid * per_w
        pltpu.sync_copy(idx_hbm.at[wid], idx_v)
        copies = []
        for j in range(nch):
            copies.append(
                pltpu.async_copy(
                    table_hbm.at[idx_v.at[j]],
                    rows_v.at[pl.ds(j * CH, CH)],
                    sem,
                )
            )
        for c in copies:
            c.wait()
        pltpu.sync_copy(rows_v, out_hbm.at[pl.ds(base, per_w)])

    return k(table, idx3)


def _tc_segment_matmul(bounds, r_s, ht_rows, tm, rel, nt):
    def body(bounds_ref, r_ref, ht_ref, tm_ref, rel_ref, out_ref):
        i = pl.program_id(0)
        lo = bounds_ref[i, 0]
        hi = bounds_ref[i, 1]
        v = ht_ref[0, 0] - ht_ref[1, 0]
        rt = r_ref[0]

        def step(rr, acc):
            m = tm_ref[rr]
            rl = rel_ref[rr]
            prod = lax.dot_general(
                v, m, (((1,), (1,)), ((), ())),
                preferred_element_type=jnp.float32,
            )
            return acc + jnp.where(rt == rr, prod + rl, 0.0)

        out_ref[...] = lax.fori_loop(
            lo, hi + 1, step, jnp.zeros((TB, D), jnp.float32))

    grid_spec = pltpu.PrefetchScalarGridSpec(
        num_scalar_prefetch=1,
        grid=(nt,),
        in_specs=[
            pl.BlockSpec((1, TB, 1), lambda i, s: (i, 0, 0)),
            pl.BlockSpec((2, 1, TB, D), lambda i, s: (0, i, 0, 0)),
            pl.BlockSpec((tm.shape[0], D, D), lambda i, s: (0, 0, 0)),
            pl.BlockSpec((rel.shape[0], 1, D), lambda i, s: (0, 0, 0)),
        ],
        out_specs=pl.BlockSpec((TB, D), lambda i, s: (i, 0)),
    )
    return pl.pallas_call(
        body,
        grid_spec=grid_spec,
        out_shape=jax.ShapeDtypeStruct((nt * TB, D), jnp.float32),
    )(bounds, r_s, ht_rows, tm, rel)


def kernel(h, t, r, ent_embeddings, rel_embeddings, transfer_matrix):
    b = h.shape[0]
    nt = b // TB
    h = h.astype(jnp.int32)
    t = t.astype(jnp.int32)
    r = r.astype(jnp.int32)

    order = jnp.argsort(r)
    r_s = jnp.take(r, order)
    h_s = jnp.take(h, order)
    t_s = jnp.take(t, order)
    inv = jnp.zeros((b,), jnp.int32).at[order].set(
        jnp.arange(b, dtype=jnp.int32))
    bounds = jnp.stack([r_s[::TB], r_s[TB - 1::TB]], axis=1)

    ent = ent_embeddings.reshape(ent_embeddings.shape[0], D)
    rel = rel_embeddings.reshape(rel_embeddings.shape[0], 1, D)
    tm = transfer_matrix

    ht_idx = jnp.concatenate([h_s, t_s])
    ht_rows = _sc_gather(ent, ht_idx).reshape(2, nt, TB, D)

    out_s = _tc_segment_matmul(
        bounds, r_s.reshape(nt, TB, 1), ht_rows, tm, rel, nt)

    return _sc_gather(out_s, inv)
